# Initial kernel scaffold; baseline (speedup 1.0000x reference)
import jax
import jax.numpy as jnp
from jax.experimental import pallas as pl


def kernel(x, tables):
    raise NotImplementedError("write your pallas kernel here")


# speedup vs baseline: 42.5958x; 42.5958x over previous
import dataclasses
import functools

import jax
import jax.numpy as jnp
from jax import lax
from jax.experimental import pallas as pl
from jax.experimental.pallas import tpu as pltpu
from jax.experimental.pallas import tpu_sc as plsc

N_LEVELS = 16
LOG2_T = 19
TABLE_SIZE = 1 << LOG2_T
MASK = TABLE_SIZE - 1
BASE_RES = 16
B_SCALE = 1.39
RES = [int(BASE_RES * (B_SCALE ** i)) for i in range(N_LEVELS)]
P1_I32 = 2654435761 - (1 << 32)
P2_I32 = 805459861

NC, NS, L = 2, 16, 16
NW = NC * NS
P = 512
CH = P // L
D = 8


def _splat_i32(v):
    return jnp.full((L,), v, dtype=jnp.int32)


def kernel(x, tables):
    n = x.shape[0]
    xt = x.T.reshape(-1)
    tab = tables.reshape(N_LEVELS * TABLE_SIZE // 4, D)
    pts_per_tile = n // NW
    nblk = pts_per_tile // P
    mesh = plsc.VectorSubcoreMesh(core_axis_name="c", subcore_axis_name="s")
    cp = pltpu.CompilerParams()
    if "needs_layout_passes" in pltpu.CompilerParams.__dataclass_fields__:
        cp = dataclasses.replace(cp, needs_layout_passes=False)
    if "use_tc_tiling_on_sc" in pltpu.CompilerParams.__dataclass_fields__:
        cp = dataclasses.replace(cp, use_tc_tiling_on_sc=False)

    @functools.partial(
        pl.kernel,
        mesh=mesh,
        compiler_params=cp,
        out_type=jax.ShapeDtypeStruct((n, 2 * N_LEVELS), jnp.float32),
        scratch_types=[
            pltpu.VMEM((3 * P,), jnp.float32),
            pltpu.VMEM((3 * P,), jnp.float32),
            pltpu.VMEM((8 * P,), jnp.int32),
            pltpu.VMEM((8 * P,), jnp.int32),
            pltpu.VMEM((8 * P, D), jnp.float32),
            pltpu.VMEM((P, 2 * N_LEVELS), jnp.float32),
            pltpu.SemaphoreType.DMA,
        ],
    )
    def sc_kernel(xt_hbm, tab_hbm, out_hbm, xv, wv, idxv, offv, rowsv, outv, sem):
        wid = lax.axis_index("s") * NC + lax.axis_index("c")
        iota = lax.iota(jnp.int32, L)

        @pl.loop(0, nblk)
        def _blk(blk):
            gbase = wid * pts_per_tile + blk * P
            for d in range(3):
                pltpu.sync_copy(xt_hbm.at[pl.ds(d * n + gbase, P)],
                                xv.at[pl.ds(d * P, P)])

            for lvl in range(N_LEVELS):
                res = float(RES[lvl])
                base = lvl * TABLE_SIZE

                @pl.loop(0, CH)
                def _hash(ch, res=res, base=base):
                    p0 = ch * L
                    vi = []
                    for d in range(3):
                        xf = xv[pl.ds(d * P + p0, L)] * res
                        vid = xf.astype(jnp.int32)
                        vid = jnp.where(vid.astype(jnp.float32) > xf, vid - 1, vid)
                        wv[pl.ds(d * P + p0, L)] = xf - vid.astype(jnp.float32)
                        vi.append(vid)
                    hx0 = vi[0]
                    hx1 = vi[0] + 1
                    hy0 = vi[1] * P1_I32
                    hy1 = hy0 + P1_I32
                    hz0 = vi[2] * P2_I32
                    hz1 = hz0 + P2_I32
                    a = [hx0 ^ hy0, hx0 ^ hy1, hx1 ^ hy0, hx1 ^ hy1]
                    for c in range(8):
                        i, j, k = c >> 2, (c >> 1) & 1, c & 1
                        h = (a[2 * i + j] ^ (hz1 if k else hz0)) & MASK | base
                        idxv[pl.ds(c * P + p0, L)] = h >> 2
                        offv[pl.ds(c * P + p0, L)] = (h & 3) << 1

                pltpu.async_copy(tab_hbm.at[idxv], rowsv, sem).wait()

                @pl.loop(0, CH)
                def _interp(ch, lvl=lvl):
                    p0 = ch * L
                    pvec = p0 + iota
                    w0 = wv[pl.ds(p0, L)]
                    w1 = wv[pl.ds(P + p0, L)]
                    w2 = wv[pl.ds(2 * P + p0, L)]
                    u0 = 1.0 - w0
                    u1 = 1.0 - w1
                    u2 = 1.0 - w2
                    yz = [u1 * u2, u1 * w2, w1 * u2, w1 * w2]
                    w8 = [u0 * yz[0], u0 * yz[1], u0 * yz[2], u0 * yz[3],
                          w0 * yz[0], w0 * yz[1], w0 * yz[2], w0 * yz[3]]
                    acc = [None, None]
                    for c in range(8):
                        rvec = pvec + c * P
                        off2 = offv[pl.ds(c * P + p0, L)]
                        for f in range(2):
                            v = plsc.load_gather(rowsv, [rvec, off2 + f])
                            term = v * w8[c]
                            acc[f] = term if acc[f] is None else acc[f] + term
                    for f in range(2):
                        plsc.store_scatter(outv, [pvec, _splat_i32(2 * lvl + f)],
                                           acc[f])

            pltpu.sync_copy(outv, out_hbm.at[pl.ds(gbase, P)])

    return sc_kernel(xt, tab)
